# baseline (speedup 1.0000x reference)
import jax
import jax.numpy as jnp
from jax import lax
from jax.experimental import pallas as pl
from jax.experimental.pallas import tpu as pltpu
from jax.experimental.pallas import tpu_sc as plsc

N1, N2 = 5000, 2500
D = 128
NC, NS, LANES = 2, 16, 16
NW = NC * NS
C = 128
NTAB0 = 5120
NACC = 2560


def _contract(a, b):
    return lax.dot_general(a, b, (((1,), (1,)), ((), ())),
                           preferred_element_type=jnp.float32)


def _combine(acc_ref, cnt_ref):
    agg = acc_ref[0, :N2, :] + acc_ref[1, :N2, :]
    cnt = cnt_ref[0, :N2, :] + cnt_ref[1, :N2, :]
    return agg / jnp.maximum(cnt, 1.0)


def _tc_mid_body(x_ref, acc_ref, cnt_ref, wl0_ref, wr0_ref, b0_ref,
                 wr1_ref, b1_ref, h_ref, base1_ref):
    mean = _combine(acc_ref, cnt_ref)
    h = jnp.maximum(_contract(mean, wl0_ref[...])
                    + _contract(x_ref[:N2], wr0_ref[...]) + b0_ref[...], 0.0)
    h_ref[...] = h
    base1_ref[...] = _contract(h, wr1_ref[...]) + b1_ref[...]


def _tc_post_body(acc_ref, cnt_ref, base_ref, wl1_ref, out_ref):
    o = _contract(_combine(acc_ref, cnt_ref), wl1_ref[...]) + base_ref[...]
    m = jnp.max(o, axis=1, keepdims=True)
    s = o - m
    lse = jnp.log(jnp.sum(jnp.exp(s), axis=1, keepdims=True))
    out_ref[...] = s - lse


def _sc_compact_call(src2d, dst2d, rows_w, rows_c, keep_n, dump_row):
    mesh = plsc.VectorSubcoreMesh(core_axis_name="c", subcore_axis_name="s",
                                  num_cores=NC)
    assert rows_w % 8 == 0 and rows_c % 8 == 0 and rows_c >= rows_w + 3

    def body(src_h, dst_h, srcc_h, dstc_h, npair_h,
             src_v, dst_v, src_c, dst_c, npv):
        cid = lax.axis_index("c")
        sid = lax.axis_index("s")
        wid = sid * NC + cid
        base = wid * rows_w
        pltpu.sync_copy(src_h.at[pl.ds(base, rows_w)], src_v)
        pltpu.sync_copy(dst_h.at[pl.ds(base, rows_w)], dst_v)

        @pl.loop(0, rows_w, init_carry=jnp.int32(0))
        def noff(r, off):
            for j in range(C // LANES):
                cs = pl.ds(j * LANES, LANES)
                d = dst_v[r, cs]
                s = src_v[r, cs]
                m = d < keep_n
                mi = m.astype(jnp.int32)
                incl = plsc.cumsum(mi)
                pos = jnp.full((LANES,), off, jnp.int32) + incl - mi
                prow = jax.lax.shift_right_logical(pos, 7)
                pcol = jax.lax.bitwise_and(pos, C - 1)
                plsc.store_scatter(src_c, [prow, pcol], s, mask=m)
                plsc.store_scatter(dst_c, [prow, pcol], d, mask=m)
                off = off + incl[LANES - 1]
            return off

        iota = lax.iota(jnp.int32, LANES)
        for k in range(4 * (C // LANES)):
            pos = jnp.full((LANES,), noff + k * LANES, jnp.int32) + iota
            prow = jax.lax.shift_right_logical(pos, 7)
            pcol = jax.lax.bitwise_and(pos, C - 1)
            plsc.store_scatter(src_c, [prow, pcol],
                               jnp.zeros((LANES,), jnp.int32))
            plsc.store_scatter(dst_c, [prow, pcol],
                               jnp.full((LANES,), dump_row, jnp.int32))

        n_pairs = jax.lax.shift_right_logical(noff + 2 * C - 1, 8)
        npv[pl.ds(0, LANES)] = jnp.full((LANES,), n_pairs, jnp.int32)
        pltpu.sync_copy(src_c, srcc_h.at[pl.ds(wid * rows_c, rows_c)])
        pltpu.sync_copy(dst_c, dstc_h.at[pl.ds(wid * rows_c, rows_c)])
        pltpu.sync_copy(npv, npair_h.at[pl.ds(wid * LANES, LANES)])

    fn = pl.kernel(
        body,
        out_type=(jax.ShapeDtypeStruct((NW * rows_c, C), jnp.int32),
                  jax.ShapeDtypeStruct((NW * rows_c, C), jnp.int32),
                  jax.ShapeDtypeStruct((NW * LANES,), jnp.int32)),
        mesh=mesh,
        compiler_params=pltpu.CompilerParams(needs_layout_passes=False),
        scratch_types=(
            pltpu.VMEM((rows_w, C), jnp.int32),
            pltpu.VMEM((rows_w, C), jnp.int32),
            pltpu.VMEM((rows_c, C), jnp.int32),
            pltpu.VMEM((rows_c, C), jnp.int32),
            pltpu.VMEM((LANES,), jnp.int32),
        ),
    )
    return fn(src2d, dst2d)


def _sc_agg_call(table, srcc, dstc, npair, npad_tab, rows_c):
    rows_t = NACC // NS
    tab_t = npad_tab // NS
    mesh = plsc.VectorSubcoreMesh(core_axis_name="c", subcore_axis_name="s",
                                  num_cores=NC)
    assert rows_t % 32 == 0 and tab_t % 8 == 0 and rows_c % 8 == 0

    def body(table_h, src_h, dst_h, npair_h, acc_h, cnt_h,
             src_c, dst_c, npv, buf_a, buf_b, ones_v, cnt_v,
             table_sh, acc_sh, cnt_sh, sem_a, sem_b, sem_ra, sem_rb, sem_c):
        cid = lax.axis_index("c")
        sid = lax.axis_index("s")
        wid = sid * NC + cid
        pltpu.sync_copy(src_h.at[pl.ds(wid * rows_c, rows_c)], src_c)
        pltpu.sync_copy(dst_h.at[pl.ds(wid * rows_c, rows_c)], dst_c)
        pltpu.sync_copy(npair_h.at[pl.ds(wid * LANES, LANES)], npv)
        for j in range(D // LANES):
            ones_v[pl.ds(j * LANES, LANES)] = jnp.ones((LANES,), jnp.float32)

        @pl.loop(0, C)
        def _zbuf(i):
            for j in range(D // LANES):
                buf_a[i, pl.ds(j * LANES, LANES)] = jnp.zeros((LANES,),
                                                              jnp.float32)

        @pl.loop(0, rows_t // 32)
        def _zero(r):
            pltpu.sync_copy(buf_a.at[pl.ds(0, 32)],
                            acc_sh.at[pl.ds(sid * rows_t + r * 32, 32)])
            pltpu.sync_copy(buf_a.at[0, pl.ds(0, 32)],
                            cnt_sh.at[pl.ds(sid * rows_t + r * 32, 32)])

        tsl = pl.ds(sid * tab_t, tab_t)
        pltpu.sync_copy(table_h.at[tsl], table_sh.at[tsl])
        plsc.subcore_barrier()

        n_pairs = npv[pl.ds(0, LANES)][0]

        def gather(ch, buf, sem):
            return pltpu.make_async_copy(table_sh.at[src_c.at[ch]], buf, sem)

        gather(0, buf_a, sem_a).start()
        gather(1, buf_b, sem_b).start()

        @pl.loop(0, n_pairs)
        def _step(jj):
            c0 = 2 * jj
            gather(c0, buf_a, sem_a).wait()
            ra = pltpu.async_copy(buf_a, acc_sh.at[dst_c.at[c0]], sem_ra,
                                  add=True)
            ca = pltpu.async_copy(ones_v, cnt_sh.at[dst_c.at[c0]], sem_c,
                                  add=True)
            gather(c0 + 1, buf_b, sem_b).wait()
            rb = pltpu.async_copy(buf_b, acc_sh.at[dst_c.at[c0 + 1]], sem_rb,
                                  add=True)
            cb = pltpu.async_copy(ones_v, cnt_sh.at[dst_c.at[c0 + 1]], sem_c,
                                  add=True)
            ra.wait()
            gather(c0 + 2, buf_a, sem_a).start()
            rb.wait()
            gather(c0 + 3, buf_b, sem_b).start()
            ca.wait()
            cb.wait()

        gather(0, buf_a, sem_a).wait()
        gather(0, buf_b, sem_b).wait()
        plsc.subcore_barrier()
        sl = pl.ds(sid * rows_t, rows_t)
        pltpu.sync_copy(acc_sh.at[sl], acc_h.at[cid, sl])
        pltpu.sync_copy(cnt_sh.at[sl], cnt_v)
        pltpu.sync_copy(cnt_v,
                        cnt_h.at[pl.ds(cid * NACC + sid * rows_t, rows_t)])

    fn = pl.kernel(
        body,
        out_type=(jax.ShapeDtypeStruct((NC, NACC, D), jnp.float32),
                  jax.ShapeDtypeStruct((NC * NACC,), jnp.float32)),
        mesh=mesh,
        scratch_types=(
            pltpu.VMEM((rows_c, C), jnp.int32),
            pltpu.VMEM((rows_c, C), jnp.int32),
            pltpu.VMEM((LANES,), jnp.int32),
            pltpu.VMEM((C, D), jnp.float32),
            pltpu.VMEM((C, D), jnp.float32),
            pltpu.VMEM((C,), jnp.float32),
            pltpu.VMEM((rows_t,), jnp.float32),
            pltpu.VMEM_SHARED((npad_tab, D), jnp.float32),
            pltpu.VMEM_SHARED((NACC, D), jnp.float32),
            pltpu.VMEM_SHARED((NACC,), jnp.float32),
            pltpu.SemaphoreType.DMA,
            pltpu.SemaphoreType.DMA,
            pltpu.SemaphoreType.DMA,
            pltpu.SemaphoreType.DMA,
            pltpu.SemaphoreType.DMA,
        ),
    )
    return fn(table, srcc, dstc, npair)


def _pad_edges(edge_index, n_edges, rows_total, dump_row):
    pad = rows_total * C - n_edges
    src = jnp.concatenate([edge_index[0], jnp.zeros((pad,), jnp.int32)])
    dst = jnp.concatenate([edge_index[1],
                           jnp.full((pad,), dump_row, jnp.int32)])
    return src.reshape(rows_total, C), dst.reshape(rows_total, C)


def kernel(x, edge_index0, edge_index1, W_l0, b_l0, W_r0, b_r0,
           W_l1, b_l1, W_r1, b_r1):
    E0 = edge_index0.shape[1]
    E1 = edge_index1.shape[1]
    rows_w0 = -(-E0 // (NW * C * 8)) * 8
    rows_w1 = -(-E1 // (NW * C * 8)) * 8
    rows_c0 = rows_w0 + 8
    rows_c1 = rows_w1 + 8
    src0, dst0 = _pad_edges(edge_index0, E0, NW * rows_w0, NACC - 1)
    src1, dst1 = _pad_edges(edge_index1, E1, NW * rows_w1, NACC - 1)

    bsum0 = (b_l0 + b_r0).reshape(1, D)
    bsum1 = (b_l1 + b_r1).reshape(1, D)

    srcc0, dstc0, npair0 = _sc_compact_call(src0, dst0, rows_w0, rows_c0,
                                            N2, NACC - 1)
    x_pad = jnp.pad(x[:N1], ((0, NTAB0 - N1), (0, 0)))
    acc0, cnt0 = _sc_agg_call(x_pad, srcc0, dstc0, npair0, NTAB0, rows_c0)
    cnt0 = cnt0.reshape(NC, NACC, 1)

    h, base1 = pl.pallas_call(
        _tc_mid_body,
        out_shape=(jax.ShapeDtypeStruct((N2, D), jnp.float32),
                   jax.ShapeDtypeStruct((N2, D), jnp.float32)),
    )(x[:N2], acc0, cnt0, W_l0, W_r0, bsum0, W_r1, bsum1)

    src1b = jnp.pad(src1.reshape(NW, rows_w1, C),
                    ((0, 0), (0, rows_c1 - rows_w1), (0, 0))
                    ).reshape(NW * rows_c1, C)
    dst1b = jnp.pad(dst1.reshape(NW, rows_w1, C),
                    ((0, 0), (0, rows_c1 - rows_w1), (0, 0)),
                    constant_values=NACC - 1).reshape(NW * rows_c1, C)
    npair1 = jnp.full((NW * LANES,), rows_w1 // 2, jnp.int32)

    h_pad = jnp.pad(h, ((0, NACC - N2), (0, 0)))
    acc1, cnt1 = _sc_agg_call(h_pad, src1b, dst1b, npair1, NACC, rows_c1)
    cnt1 = cnt1.reshape(NC, NACC, 1)

    out = pl.pallas_call(
        _tc_post_body,
        out_shape=jax.ShapeDtypeStruct((N2, D), jnp.float32),
    )(acc1, cnt1, base1, W_l1)
    return out


# speedup vs baseline: 1.0396x; 1.0396x over previous
import jax
import jax.numpy as jnp
from jax import lax
from jax.experimental import pallas as pl
from jax.experimental.pallas import tpu as pltpu
from jax.experimental.pallas import tpu_sc as plsc

N1, N2 = 5000, 2500
D = 128
NC, NS, LANES = 2, 16, 16
NW = NC * NS
C = 128
NTAB0 = 5120
NACC = 2560


def _contract(a, b):
    return lax.dot_general(a, b, (((1,), (1,)), ((), ())),
                           preferred_element_type=jnp.float32)


def _combine(acc_ref, cnt_ref):
    agg = acc_ref[0, :N2, :] + acc_ref[1, :N2, :]
    cnt = cnt_ref[0, :N2, :] + cnt_ref[1, :N2, :]
    return agg / jnp.maximum(cnt, 1.0)


def _tc_mid_body(x_ref, acc_ref, cnt_ref, wl0_ref, wr0_ref, b0_ref,
                 wr1_ref, b1_ref, h_ref, base1_ref):
    mean = _combine(acc_ref, cnt_ref)
    h = jnp.maximum(_contract(mean, wl0_ref[...])
                    + _contract(x_ref[:N2], wr0_ref[...]) + b0_ref[...], 0.0)
    h_ref[...] = h
    base1_ref[...] = _contract(h, wr1_ref[...]) + b1_ref[...]


def _tc_post_body(acc_ref, cnt_ref, base_ref, wl1_ref, out_ref):
    o = _contract(_combine(acc_ref, cnt_ref), wl1_ref[...]) + base_ref[...]
    m = jnp.max(o, axis=1, keepdims=True)
    s = o - m
    lse = jnp.log(jnp.sum(jnp.exp(s), axis=1, keepdims=True))
    out_ref[...] = s - lse


def _sc_compact_call(src2d, dst2d, rows_w, rows_c, keep_n, dump_row):
    mesh = plsc.VectorSubcoreMesh(core_axis_name="c", subcore_axis_name="s",
                                  num_cores=NC)
    assert rows_w % 8 == 0 and rows_c % 8 == 0 and rows_c >= rows_w + 3

    def body(src_h, dst_h, srcc_h, dstc_h, npair_h,
             src_v, dst_v, src_c, dst_c, npv):
        cid = lax.axis_index("c")
        sid = lax.axis_index("s")
        wid = sid * NC + cid
        base = wid * rows_w
        pltpu.sync_copy(src_h.at[pl.ds(base, rows_w)], src_v)
        pltpu.sync_copy(dst_h.at[pl.ds(base, rows_w)], dst_v)

        @pl.loop(0, rows_w, init_carry=jnp.int32(0))
        def noff(r, off):
            for j in range(C // LANES):
                cs = pl.ds(j * LANES, LANES)
                d = dst_v[r, cs]
                s = src_v[r, cs]
                m = d < keep_n
                mi = m.astype(jnp.int32)
                incl = plsc.cumsum(mi)
                pos = jnp.full((LANES,), off, jnp.int32) + incl - mi
                prow = jax.lax.shift_right_logical(pos, 7)
                pcol = jax.lax.bitwise_and(pos, C - 1)
                plsc.store_scatter(src_c, [prow, pcol], s, mask=m)
                plsc.store_scatter(dst_c, [prow, pcol], d, mask=m)
                off = off + incl[LANES - 1]
            return off

        iota = lax.iota(jnp.int32, LANES)
        for k in range(4 * (C // LANES)):
            pos = jnp.full((LANES,), noff + k * LANES, jnp.int32) + iota
            prow = jax.lax.shift_right_logical(pos, 7)
            pcol = jax.lax.bitwise_and(pos, C - 1)
            plsc.store_scatter(src_c, [prow, pcol],
                               jnp.zeros((LANES,), jnp.int32))
            plsc.store_scatter(dst_c, [prow, pcol],
                               jnp.full((LANES,), dump_row, jnp.int32))

        n_pairs = jax.lax.shift_right_logical(noff + 2 * C - 1, 8)
        npv[pl.ds(0, LANES)] = jnp.full((LANES,), n_pairs, jnp.int32)
        pltpu.sync_copy(src_c, srcc_h.at[pl.ds(wid * rows_c, rows_c)])
        pltpu.sync_copy(dst_c, dstc_h.at[pl.ds(wid * rows_c, rows_c)])
        pltpu.sync_copy(npv, npair_h.at[pl.ds(wid * LANES, LANES)])

    fn = pl.kernel(
        body,
        out_type=(jax.ShapeDtypeStruct((NW * rows_c, C), jnp.int32),
                  jax.ShapeDtypeStruct((NW * rows_c, C), jnp.int32),
                  jax.ShapeDtypeStruct((NW * LANES,), jnp.int32)),
        mesh=mesh,
        compiler_params=pltpu.CompilerParams(needs_layout_passes=False),
        scratch_types=(
            pltpu.VMEM((rows_w, C), jnp.int32),
            pltpu.VMEM((rows_w, C), jnp.int32),
            pltpu.VMEM((rows_c, C), jnp.int32),
            pltpu.VMEM((rows_c, C), jnp.int32),
            pltpu.VMEM((LANES,), jnp.int32),
        ),
    )
    return fn(src2d, dst2d)


def _sc_agg_call(table, srcc, dstc, npair, npad_tab, rows_c):
    rows_t = NACC // NS
    tab_t = npad_tab // NS
    mesh = plsc.VectorSubcoreMesh(core_axis_name="c", subcore_axis_name="s",
                                  num_cores=NC)
    assert rows_t % 32 == 0 and tab_t % 8 == 0 and rows_c % 8 == 0

    def body(table_h, src_h, dst_h, npair_h, acc_h, cnt_h,
             src_c, dst_c, npv, buf_a, buf_b, ones_v, cnt_v,
             table_sh, acc_sh, cnt_sh, sem_a, sem_b, sem_ra, sem_rb, sem_c):
        cid = lax.axis_index("c")
        sid = lax.axis_index("s")
        wid = sid * NC + cid
        pltpu.sync_copy(src_h.at[pl.ds(wid * rows_c, rows_c)], src_c)
        pltpu.sync_copy(dst_h.at[pl.ds(wid * rows_c, rows_c)], dst_c)
        pltpu.sync_copy(npair_h.at[pl.ds(wid * LANES, LANES)], npv)
        for j in range(D // LANES):
            ones_v[pl.ds(j * LANES, LANES)] = jnp.ones((LANES,), jnp.float32)

        @pl.loop(0, C)
        def _zbuf(i):
            for j in range(D // LANES):
                buf_a[i, pl.ds(j * LANES, LANES)] = jnp.zeros((LANES,),
                                                              jnp.float32)

        @pl.loop(0, rows_t // 32)
        def _zero(r):
            pltpu.sync_copy(buf_a.at[pl.ds(0, 32)],
                            acc_sh.at[pl.ds(sid * rows_t + r * 32, 32)])
            pltpu.sync_copy(buf_a.at[0, pl.ds(0, 32)],
                            cnt_sh.at[pl.ds(sid * rows_t + r * 32, 32)])

        tsl = pl.ds(sid * tab_t, tab_t)
        pltpu.sync_copy(table_h.at[tsl], table_sh.at[tsl])
        plsc.subcore_barrier()

        n_pairs = npv[pl.ds(0, LANES)][0]

        def gather(ch, buf, sem):
            return pltpu.make_async_copy(table_sh.at[src_c.at[ch]], buf, sem)

        gather(0, buf_a, sem_a).start()

        @pl.loop(0, n_pairs)
        def _step(jj):
            c0 = 2 * jj
            gather(c0 + 1, buf_b, sem_b).start()
            gather(c0, buf_a, sem_a).wait()
            ca = pltpu.async_copy(ones_v, cnt_sh.at[dst_c.at[c0]], sem_c,
                                  add=True)
            pltpu.sync_copy(buf_a, acc_sh.at[dst_c.at[c0]], add=True)
            gather(c0 + 2, buf_a, sem_a).start()
            gather(c0 + 1, buf_b, sem_b).wait()
            cb = pltpu.async_copy(ones_v, cnt_sh.at[dst_c.at[c0 + 1]], sem_c,
                                  add=True)
            pltpu.sync_copy(buf_b, acc_sh.at[dst_c.at[c0 + 1]], add=True)
            ca.wait()
            cb.wait()

        gather(0, buf_a, sem_a).wait()
        plsc.subcore_barrier()
        sl = pl.ds(sid * rows_t, rows_t)
        pltpu.sync_copy(acc_sh.at[sl], acc_h.at[cid, sl])
        pltpu.sync_copy(cnt_sh.at[sl], cnt_v)
        pltpu.sync_copy(cnt_v,
                        cnt_h.at[pl.ds(cid * NACC + sid * rows_t, rows_t)])

    fn = pl.kernel(
        body,
        out_type=(jax.ShapeDtypeStruct((NC, NACC, D), jnp.float32),
                  jax.ShapeDtypeStruct((NC * NACC,), jnp.float32)),
        mesh=mesh,
        scratch_types=(
            pltpu.VMEM((rows_c, C), jnp.int32),
            pltpu.VMEM((rows_c, C), jnp.int32),
            pltpu.VMEM((LANES,), jnp.int32),
            pltpu.VMEM((C, D), jnp.float32),
            pltpu.VMEM((C, D), jnp.float32),
            pltpu.VMEM((C,), jnp.float32),
            pltpu.VMEM((rows_t,), jnp.float32),
            pltpu.VMEM_SHARED((npad_tab, D), jnp.float32),
            pltpu.VMEM_SHARED((NACC, D), jnp.float32),
            pltpu.VMEM_SHARED((NACC,), jnp.float32),
            pltpu.SemaphoreType.DMA,
            pltpu.SemaphoreType.DMA,
            pltpu.SemaphoreType.DMA,
            pltpu.SemaphoreType.DMA,
            pltpu.SemaphoreType.DMA,
        ),
    )
    return fn(table, srcc, dstc, npair)


def _pad_edges(edge_index, n_edges, rows_total, dump_row):
    pad = rows_total * C - n_edges
    src = jnp.concatenate([edge_index[0], jnp.zeros((pad,), jnp.int32)])
    dst = jnp.concatenate([edge_index[1],
                           jnp.full((pad,), dump_row, jnp.int32)])
    return src.reshape(rows_total, C), dst.reshape(rows_total, C)


def kernel(x, edge_index0, edge_index1, W_l0, b_l0, W_r0, b_r0,
           W_l1, b_l1, W_r1, b_r1):
    E0 = edge_index0.shape[1]
    E1 = edge_index1.shape[1]
    rows_w0 = -(-E0 // (NW * C * 8)) * 8
    rows_w1 = -(-E1 // (NW * C * 8)) * 8
    rows_c0 = rows_w0 + 8
    rows_c1 = rows_w1 + 8
    src0, dst0 = _pad_edges(edge_index0, E0, NW * rows_w0, NACC - 1)
    src1, dst1 = _pad_edges(edge_index1, E1, NW * rows_w1, NACC - 1)

    bsum0 = (b_l0 + b_r0).reshape(1, D)
    bsum1 = (b_l1 + b_r1).reshape(1, D)

    srcc0, dstc0, npair0 = _sc_compact_call(src0, dst0, rows_w0, rows_c0,
                                            N2, NACC - 1)
    x_pad = jnp.pad(x[:N1], ((0, NTAB0 - N1), (0, 0)))
    acc0, cnt0 = _sc_agg_call(x_pad, srcc0, dstc0, npair0, NTAB0, rows_c0)
    cnt0 = cnt0.reshape(NC, NACC, 1)

    h, base1 = pl.pallas_call(
        _tc_mid_body,
        out_shape=(jax.ShapeDtypeStruct((N2, D), jnp.float32),
                   jax.ShapeDtypeStruct((N2, D), jnp.float32)),
    )(x[:N2], acc0, cnt0, W_l0, W_r0, bsum0, W_r1, bsum1)

    src1b = jnp.pad(src1.reshape(NW, rows_w1, C),
                    ((0, 0), (0, rows_c1 - rows_w1), (0, 0))
                    ).reshape(NW * rows_c1, C)
    dst1b = jnp.pad(dst1.reshape(NW, rows_w1, C),
                    ((0, 0), (0, rows_c1 - rows_w1), (0, 0)),
                    constant_values=NACC - 1).reshape(NW * rows_c1, C)
    npair1 = jnp.full((NW * LANES,), rows_w1 // 2, jnp.int32)

    h_pad = jnp.pad(h, ((0, NACC - N2), (0, 0)))
    acc1, cnt1 = _sc_agg_call(h_pad, src1b, dst1b, npair1, NACC, rows_c1)
    cnt1 = cnt1.reshape(NC, NACC, 1)

    out = pl.pallas_call(
        _tc_post_body,
        out_shape=jax.ShapeDtypeStruct((N2, D), jnp.float32),
    )(acc1, cnt1, base1, W_l1)
    return out


# speedup vs baseline: 1.0417x; 1.0021x over previous
import jax
import jax.numpy as jnp
from jax import lax
from jax.experimental import pallas as pl
from jax.experimental.pallas import tpu as pltpu
from jax.experimental.pallas import tpu_sc as plsc

N1, N2 = 5000, 2500
D = 128
NC, NS, LANES = 2, 16, 16
NW = NC * NS
C = 128
NTAB0 = 5120
NACC = 2560


def _contract(a, b):
    return lax.dot_general(a, b, (((1,), (1,)), ((), ())),
                           preferred_element_type=jnp.float32)


def _combine(acc_ref, cnt_ref):
    agg = acc_ref[0, :N2, :] + acc_ref[1, :N2, :]
    cnt = cnt_ref[0, :N2, :] + cnt_ref[1, :N2, :]
    return agg / jnp.maximum(cnt, 1.0)


def _tc_mid_body(x_ref, acc_ref, cnt_ref, wl0_ref, wr0_ref, b0_ref,
                 wr1_ref, b1_ref, h_ref, base1_ref):
    mean = _combine(acc_ref, cnt_ref)
    h = jnp.maximum(_contract(mean, wl0_ref[...])
                    + _contract(x_ref[:N2], wr0_ref[...]) + b0_ref[...], 0.0)
    h_ref[...] = h
    base1_ref[...] = _contract(h, wr1_ref[...]) + b1_ref[...]


def _tc_post_body(acc_ref, cnt_ref, base_ref, wl1_ref, out_ref):
    o = _contract(_combine(acc_ref, cnt_ref), wl1_ref[...]) + base_ref[...]
    m = jnp.max(o, axis=1, keepdims=True)
    s = o - m
    lse = jnp.log(jnp.sum(jnp.exp(s), axis=1, keepdims=True))
    out_ref[...] = s - lse


def _sc_compact_call(src2d, dst2d, rows_w, rows_c, keep_n, dump_row):
    mesh = plsc.VectorSubcoreMesh(core_axis_name="c", subcore_axis_name="s",
                                  num_cores=NC)
    assert rows_w % 8 == 0 and rows_c % 8 == 0 and rows_c >= rows_w + 3

    def body(src_h, dst_h, srcc_h, dstc_h, npair_h,
             src_v, dst_v, src_c, dst_c, npv):
        cid = lax.axis_index("c")
        sid = lax.axis_index("s")
        wid = sid * NC + cid
        base = wid * rows_w
        pltpu.sync_copy(src_h.at[pl.ds(base, rows_w)], src_v)
        pltpu.sync_copy(dst_h.at[pl.ds(base, rows_w)], dst_v)

        @pl.loop(0, rows_w, init_carry=jnp.int32(0))
        def noff(r, off):
            for j in range(C // LANES):
                cs = pl.ds(j * LANES, LANES)
                d = dst_v[r, cs]
                s = src_v[r, cs]
                m = d < keep_n
                mi = m.astype(jnp.int32)
                incl = plsc.cumsum(mi)
                pos = jnp.full((LANES,), off, jnp.int32) + incl - mi
                prow = jax.lax.shift_right_logical(pos, 7)
                pcol = jax.lax.bitwise_and(pos, C - 1)
                plsc.store_scatter(src_c, [prow, pcol], s, mask=m)
                plsc.store_scatter(dst_c, [prow, pcol], d, mask=m)
                off = off + incl[LANES - 1]
            return off

        iota = lax.iota(jnp.int32, LANES)
        for k in range(4 * (C // LANES)):
            pos = jnp.full((LANES,), noff + k * LANES, jnp.int32) + iota
            prow = jax.lax.shift_right_logical(pos, 7)
            pcol = jax.lax.bitwise_and(pos, C - 1)
            plsc.store_scatter(src_c, [prow, pcol],
                               jnp.zeros((LANES,), jnp.int32))
            plsc.store_scatter(dst_c, [prow, pcol],
                               jnp.full((LANES,), dump_row, jnp.int32))

        n_pairs = jax.lax.shift_right_logical(noff + 2 * C - 1, 8)
        npv[pl.ds(0, LANES)] = jnp.full((LANES,), n_pairs, jnp.int32)
        pltpu.sync_copy(src_c, srcc_h.at[pl.ds(wid * rows_c, rows_c)])
        pltpu.sync_copy(dst_c, dstc_h.at[pl.ds(wid * rows_c, rows_c)])
        pltpu.sync_copy(npv, npair_h.at[pl.ds(wid * LANES, LANES)])

    fn = pl.kernel(
        body,
        out_type=(jax.ShapeDtypeStruct((NW * rows_c, C), jnp.int32),
                  jax.ShapeDtypeStruct((NW * rows_c, C), jnp.int32),
                  jax.ShapeDtypeStruct((NW * LANES,), jnp.int32)),
        mesh=mesh,
        compiler_params=pltpu.CompilerParams(needs_layout_passes=False),
        scratch_types=(
            pltpu.VMEM((rows_w, C), jnp.int32),
            pltpu.VMEM((rows_w, C), jnp.int32),
            pltpu.VMEM((rows_c, C), jnp.int32),
            pltpu.VMEM((rows_c, C), jnp.int32),
            pltpu.VMEM((LANES,), jnp.int32),
        ),
    )
    return fn(src2d, dst2d)


def _sc_agg_call(table, srcc, dstc, npair, npad_tab, rows_c):
    rows_t = NACC // NS
    tab_t = npad_tab // NS
    mesh = plsc.VectorSubcoreMesh(core_axis_name="c", subcore_axis_name="s",
                                  num_cores=NC)
    assert rows_t % 32 == 0 and tab_t % 8 == 0 and rows_c % 8 == 0

    def body(table_h, src_h, dst_h, npair_h, acc_h, cnt_h,
             src_c, dst_c, npv, buf_a, buf_b, ones_v, cnt_v,
             table_sh, acc_sh, cnt_sh, sem_a, sem_b, sem_c):
        cid = lax.axis_index("c")
        sid = lax.axis_index("s")
        wid = sid * NC + cid
        pltpu.sync_copy(src_h.at[pl.ds(wid * rows_c, rows_c)], src_c)
        pltpu.sync_copy(dst_h.at[pl.ds(wid * rows_c, rows_c)], dst_c)
        pltpu.sync_copy(npair_h.at[pl.ds(wid * LANES, LANES)], npv)
        for j in range(D // LANES):
            ones_v[pl.ds(j * LANES, LANES)] = jnp.ones((LANES,), jnp.float32)

        @pl.loop(0, C)
        def _zbuf(i):
            for j in range(D // LANES):
                buf_a[i, pl.ds(j * LANES, LANES)] = jnp.zeros((LANES,),
                                                              jnp.float32)

        @pl.loop(0, rows_t // 32)
        def _zero(r):
            pltpu.sync_copy(buf_a.at[pl.ds(0, 32)],
                            acc_sh.at[pl.ds(sid * rows_t + r * 32, 32)])
            pltpu.sync_copy(buf_a.at[0, pl.ds(0, 32)],
                            cnt_sh.at[pl.ds(sid * rows_t + r * 32, 32)])

        tsl = pl.ds(sid * tab_t, tab_t)
        pltpu.sync_copy(table_h.at[tsl], table_sh.at[tsl])
        plsc.subcore_barrier()

        n_pairs = npv[pl.ds(0, LANES)][0]

        def gather(ch, buf, sem):
            return pltpu.make_async_copy(table_sh.at[src_c.at[ch]], buf, sem)

        gather(0, buf_a, sem_a).start()

        @pl.loop(0, n_pairs)
        def _step(jj):
            c0 = 2 * jj
            gather(c0 + 1, buf_b, sem_b).start()
            gather(c0, buf_a, sem_a).wait()
            ca = pltpu.async_copy(ones_v, cnt_sh.at[dst_c.at[c0]], sem_c,
                                  add=True)
            pltpu.sync_copy(buf_a, acc_sh.at[dst_c.at[c0]], add=True)
            gather(c0 + 2, buf_a, sem_a).start()
            gather(c0 + 1, buf_b, sem_b).wait()
            cb = pltpu.async_copy(ones_v, cnt_sh.at[dst_c.at[c0 + 1]], sem_c,
                                  add=True)
            pltpu.sync_copy(buf_b, acc_sh.at[dst_c.at[c0 + 1]], add=True)
            ca.wait()
            cb.wait()

        gather(0, buf_a, sem_a).wait()
        plsc.subcore_barrier()
        sl = pl.ds(sid * rows_t, rows_t)
        pltpu.sync_copy(acc_sh.at[sl], acc_h.at[cid, sl])
        pltpu.sync_copy(cnt_sh.at[sl], cnt_v)
        pltpu.sync_copy(cnt_v,
                        cnt_h.at[pl.ds(cid * NACC + sid * rows_t, rows_t)])

    fn = pl.kernel(
        body,
        out_type=(jax.ShapeDtypeStruct((NC, NACC, D), jnp.float32),
                  jax.ShapeDtypeStruct((NC * NACC,), jnp.float32)),
        mesh=mesh,
        scratch_types=(
            pltpu.VMEM((rows_c, C), jnp.int32),
            pltpu.VMEM((rows_c, C), jnp.int32),
            pltpu.VMEM((LANES,), jnp.int32),
            pltpu.VMEM((C, D), jnp.float32),
            pltpu.VMEM((C, D), jnp.float32),
            pltpu.VMEM((C,), jnp.float32),
            pltpu.VMEM((rows_t,), jnp.float32),
            pltpu.VMEM_SHARED((npad_tab, D), jnp.float32),
            pltpu.VMEM_SHARED((NACC, D), jnp.float32),
            pltpu.VMEM_SHARED((NACC,), jnp.float32),
            pltpu.SemaphoreType.DMA,
            pltpu.SemaphoreType.DMA,
            pltpu.SemaphoreType.DMA,
        ),
    )
    return fn(table, srcc, dstc, npair)


def _pad_edges(edge_index, n_edges, rows_total, dump_row):
    pad = rows_total * C - n_edges
    src = jnp.concatenate([edge_index[0], jnp.zeros((pad,), jnp.int32)])
    dst = jnp.concatenate([edge_index[1],
                           jnp.full((pad,), dump_row, jnp.int32)])
    return src.reshape(rows_total, C), dst.reshape(rows_total, C)


def kernel(x, edge_index0, edge_index1, W_l0, b_l0, W_r0, b_r0,
           W_l1, b_l1, W_r1, b_r1):
    E0 = edge_index0.shape[1]
    E1 = edge_index1.shape[1]
    rows_w0 = -(-E0 // (NW * C * 8)) * 8
    rows_w1 = -(-E1 // (NW * C * 8)) * 8
    rows_c0 = rows_w0 + 8
    rows_c1 = rows_w1 + 8
    src0, dst0 = _pad_edges(edge_index0, E0, NW * rows_w0, NACC - 1)
    src1, dst1 = _pad_edges(edge_index1, E1, NW * rows_w1, NACC - 1)

    bsum0 = (b_l0 + b_r0).reshape(1, D)
    bsum1 = (b_l1 + b_r1).reshape(1, D)

    srcc0, dstc0, npair0 = _sc_compact_call(src0, dst0, rows_w0, rows_c0,
                                            N2, NACC - 1)
    x_pad = jnp.pad(x[:N1], ((0, NTAB0 - N1), (0, 0)))
    acc0, cnt0 = _sc_agg_call(x_pad, srcc0, dstc0, npair0, NTAB0, rows_c0)
    cnt0 = cnt0.reshape(NC, NACC, 1)

    h, base1 = pl.pallas_call(
        _tc_mid_body,
        out_shape=(jax.ShapeDtypeStruct((N2, D), jnp.float32),
                   jax.ShapeDtypeStruct((N2, D), jnp.float32)),
    )(x[:N2], acc0, cnt0, W_l0, W_r0, bsum0, W_r1, bsum1)

    src1b = jnp.pad(src1.reshape(NW, rows_w1, C),
                    ((0, 0), (0, rows_c1 - rows_w1), (0, 0))
                    ).reshape(NW * rows_c1, C)
    dst1b = jnp.pad(dst1.reshape(NW, rows_w1, C),
                    ((0, 0), (0, rows_c1 - rows_w1), (0, 0)),
                    constant_values=NACC - 1).reshape(NW * rows_c1, C)
    npair1 = jnp.full((NW * LANES,), rows_w1 // 2, jnp.int32)

    h_pad = jnp.pad(h, ((0, NACC - N2), (0, 0)))
    acc1, cnt1 = _sc_agg_call(h_pad, src1b, dst1b, npair1, NACC, rows_c1)
    cnt1 = cnt1.reshape(NC, NACC, 1)

    out = pl.pallas_call(
        _tc_post_body,
        out_shape=jax.ShapeDtypeStruct((N2, D), jnp.float32),
    )(acc1, cnt1, base1, W_l1)
    return out
